# baseline (speedup 1.0000x reference)
import functools

import jax
import jax.numpy as jnp
from jax import lax
from jax.experimental import pallas as pl
from jax.experimental.pallas import tpu as pltpu
from jax.experimental.pallas import tpu_sc as plsc

_info = plsc.get_sparse_core_info()
_NC, _NS, _L = _info.num_cores, _info.num_subcores, _info.num_lanes
_NW = _NC * _NS

_VPI = 2
_UNROLL = 1


def _sample_kernel(chunk, u_hbm, cdf_hbm, tb_hbm, tc_hbm, out_hbm,
                   cdf_v, tb_v, tc_v, u_v, out_v):
    wid = lax.axis_index("s") * _NC + lax.axis_index("c")
    base = wid * chunk
    pltpu.sync_copy(cdf_hbm, cdf_v)
    pltpu.sync_copy(tb_hbm, tb_v)
    pltpu.sync_copy(tc_hbm, tc_v)
    pltpu.sync_copy(u_hbm.at[pl.ds(base, chunk)], u_v)

    lane = jax.lax.iota(jnp.int32, _L)
    lane2048 = lane + 2048

    def splat(i):
        return plsc.load_gather(cdf_v, [(i << 4) + lane])

    c128, c64, c192 = splat(128), splat(64), splat(192)
    c32, c96, c160, c224 = splat(32), splat(96), splat(160), splat(224)
    zero = jnp.zeros((_L,), jnp.float32)

    @plsc.parallel_loop(0, chunk // (_L * _VPI), unroll=_UNROLL)
    def body(i):
        us = [u_v[pl.ds((i * _VPI + j) * _L, _L)] for j in range(_VPI)]
        p1 = [c128 < u for u in us]
        ms = [jnp.where(p, lane2048, lane) for p in p1]
        v2 = [jnp.where(p, c192, c64) for p in p1]
        p2 = [v < u for v, u in zip(v2, us)]
        ms = [jnp.where(p, m + 1024, m) for p, m in zip(p2, ms)]
        v3 = [
            jnp.where(a, jnp.where(b_, c224, c160), jnp.where(b_, c96, c32))
            for a, b_ in zip(p1, p2)
        ]
        ms = [jnp.where(v < u, m + 512, m) for v, u, m in zip(v3, us, ms)]
        for b in (256, 128, 64, 32, 16):
            cand = [m + b for m in ms]
            vals = [plsc.load_gather(cdf_v, [c]) for c in cand]
            ms = [
                jnp.where(v < u, c, m)
                for v, u, c, m in zip(vals, us, cand, ms)
            ]
        tbs = [plsc.load_gather(tb_v, [m]) for m in ms]
        tcs = [plsc.load_gather(tc_v, [m]) for m in ms]
        for j in range(_VPI):
            out_v[pl.ds((i * _VPI + j) * _L, _L)] = tcs[j] + us[j] * tbs[j]

    pltpu.sync_copy(out_v, out_hbm.at[pl.ds(base, chunk)])


def kernel(u, pdf, cdf, func):
    del pdf
    n = func.shape[0]
    b = u.shape[0]
    chunk = b // _NW
    off = jnp.arange(n, dtype=jnp.float32)
    den = cdf[1:] - cdf[:-1]
    inv_n = jnp.float32(1.0 / n)
    tb = jnp.where(den > 0, 1.0 / jnp.where(den > 0, den, 1.0), 1.0) * inv_n
    tc = off * inv_n - cdf[:-1] * tb
    tb = jnp.repeat(jnp.concatenate([tb[1:], tb[-1:]]), 16)
    tc = jnp.repeat(jnp.concatenate([tc[1:], tc[-1:]]), 16)
    cdf_rep = jnp.repeat(cdf, 16)
    mesh = plsc.VectorSubcoreMesh(core_axis_name="c", subcore_axis_name="s")
    run = pl.kernel(
        functools.partial(_sample_kernel, chunk),
        out_type=jax.ShapeDtypeStruct((b,), jnp.float32),
        mesh=mesh,
        scratch_types=[
            pltpu.VMEM((cdf.shape[0] * 16,), jnp.float32),
            pltpu.VMEM((n * 16,), jnp.float32),
            pltpu.VMEM((n * 16,), jnp.float32),
            pltpu.VMEM((chunk,), jnp.float32),
            pltpu.VMEM((chunk,), jnp.float32),
        ],
        compiler_params=pltpu.CompilerParams(needs_layout_passes=False),
    )
    return run(u, cdf_rep, tb, tc)


# speedup vs baseline: 1.0017x; 1.0017x over previous
import functools

import jax
import jax.numpy as jnp
from jax import lax
from jax.experimental import pallas as pl
from jax.experimental.pallas import tpu as pltpu
from jax.experimental.pallas import tpu_sc as plsc

_info = plsc.get_sparse_core_info()
_NC, _NS, _L = _info.num_cores, _info.num_subcores, _info.num_lanes
_NW = _NC * _NS

_VPI = 2
_UNROLL = 1


def _sample_kernel(chunk, u_hbm, cdf_hbm, tb_hbm, tc_hbm, out_hbm,
                   cdf_v, tb_v, tc_v, u_v, out_v):
    wid = lax.axis_index("s") * _NC + lax.axis_index("c")
    base = wid * chunk
    pltpu.sync_copy(cdf_hbm, cdf_v)
    pltpu.sync_copy(tb_hbm, tb_v)
    pltpu.sync_copy(tc_hbm, tc_v)
    pltpu.sync_copy(u_hbm.at[pl.ds(base, chunk)], u_v)

    lane = jax.lax.iota(jnp.int32, _L)
    lane2048 = lane + 2048

    def splat(i):
        return plsc.load_gather(cdf_v, [(i << 4) + lane])

    c128, c64, c192 = splat(128), splat(64), splat(192)
    zero = jnp.zeros((_L,), jnp.float32)

    @plsc.parallel_loop(0, chunk // (_L * _VPI), unroll=_UNROLL)
    def body(i):
        us = [u_v[pl.ds((i * _VPI + j) * _L, _L)] for j in range(_VPI)]
        p1 = [c128 < u for u in us]
        ms = [jnp.where(p, lane2048, lane) for p in p1]
        v2 = [jnp.where(p, c192, c64) for p in p1]
        ms = [jnp.where(v < u, m + 1024, m) for v, u, m in zip(v2, us, ms)]
        for b in (512, 256, 128, 64, 32, 16):
            cand = [m + b for m in ms]
            vals = [plsc.load_gather(cdf_v, [c]) for c in cand]
            ms = [
                jnp.where(v < u, c, m)
                for v, u, c, m in zip(vals, us, cand, ms)
            ]
        tbs = [plsc.load_gather(tb_v, [m]) for m in ms]
        tcs = [plsc.load_gather(tc_v, [m]) for m in ms]
        for j in range(_VPI):
            out_v[pl.ds((i * _VPI + j) * _L, _L)] = tcs[j] + us[j] * tbs[j]

    pltpu.sync_copy(out_v, out_hbm.at[pl.ds(base, chunk)])


def kernel(u, pdf, cdf, func):
    del pdf
    n = func.shape[0]
    b = u.shape[0]
    chunk = b // _NW
    off = jnp.arange(n, dtype=jnp.float32)
    den = cdf[1:] - cdf[:-1]
    inv_n = jnp.float32(1.0 / n)
    tb = jnp.where(den > 0, 1.0 / jnp.where(den > 0, den, 1.0), 1.0) * inv_n
    tc = off * inv_n - cdf[:-1] * tb
    tb = jnp.repeat(jnp.concatenate([tb[1:], tb[-1:]]), 16)
    tc = jnp.repeat(jnp.concatenate([tc[1:], tc[-1:]]), 16)
    cdf_rep = jnp.repeat(cdf, 16)
    mesh = plsc.VectorSubcoreMesh(core_axis_name="c", subcore_axis_name="s")
    run = pl.kernel(
        functools.partial(_sample_kernel, chunk),
        out_type=jax.ShapeDtypeStruct((b,), jnp.float32),
        mesh=mesh,
        scratch_types=[
            pltpu.VMEM((cdf.shape[0] * 16,), jnp.float32),
            pltpu.VMEM((n * 16,), jnp.float32),
            pltpu.VMEM((n * 16,), jnp.float32),
            pltpu.VMEM((chunk,), jnp.float32),
            pltpu.VMEM((chunk,), jnp.float32),
        ],
        compiler_params=pltpu.CompilerParams(needs_layout_passes=False),
    )
    return run(u, cdf_rep, tb, tc)
